# Initial kernel scaffold; baseline (speedup 1.0000x reference)
import jax
import jax.numpy as jnp
from jax.experimental import pallas as pl


def kernel(x, edge_attr, params, edge_index, batch):
    raise NotImplementedError("write your pallas kernel here")


# speedup vs baseline: 10.6390x; 10.6390x over previous
import functools
import jax
import jax.numpy as jnp
from jax import lax
from jax.experimental import pallas as pl
from jax.experimental.pallas import tpu as pltpu
from jax.experimental.pallas import tpu_sc as plsc

N_, E_, G_, H_ = 10000, 320000, 256, 128
NC, NS, L = 2, 16, 16
NW = NC * NS
CH = 80
NA_N = 10112
NA_G = 384
NE_M = 10240
EPS = 1e-16
F32 = jnp.float32


def _lrelu(t):
    return jnp.where(t > 0, t, 0.01 * t)


def _elu(t):
    return jnp.where(t > 0, t, jnp.exp(jnp.minimum(t, 0.0)) - 1.0)


def _ceil128(n):
    return -(-n // 128) * 128


def _lane_reduce(x, buf, op):
    lanes = lax.iota(jnp.int32, L)
    for sh in (8, 4, 2, 1):
        buf[pl.ds(0, L)] = x
        x = op(x, plsc.load_gather(buf, [lanes ^ sh]))
    return x


def _dot(a, b):
    return jax.lax.dot_general(a, b, (((1,), (0,)), ((), ())),
                               precision=jax.lax.Precision.HIGHEST,
                               preferred_element_type=F32)


_BM = 1000


def _full(shape):
    return pl.BlockSpec(shape, lambda i: tuple(0 for _ in shape))


def _prep0(x, lin1_wt, lin1_b, w1a_t, w2_t, att_r):
    def body(x_r, wt_r, b_r, wat_r, w2t_r, ar_r, h0_r, u_r, w2_r, r_r):
        h0 = _lrelu(_dot(x_r[...], wt_r[...]) + b_r[...])
        h0_r[...] = h0
        u_r[...] = _dot(h0, wat_r[...])
        w2_r[...] = _dot(h0, w2t_r[...])
        r_r[...] = jnp.sum(h0 * ar_r[...], axis=1, keepdims=True)

    grid = (N_ // _BM,)
    bs_row = pl.BlockSpec((_BM, H_), lambda i: (i, 0))
    bs_one = pl.BlockSpec((_BM, 1), lambda i: (i, 0))
    return pl.pallas_call(
        body,
        grid=grid,
        in_specs=[bs_row, _full((H_, H_)), _full((1, H_)), _full((H_, H_)),
                  _full((H_, H_)), _full((1, H_))],
        out_specs=[bs_row, bs_row, bs_row, bs_one],
        out_shape=[jax.ShapeDtypeStruct((N_, H_), F32)] * 3
        + [jax.ShapeDtypeStruct((N_, 1), F32)],
    )(x, lin1_wt, lin1_b, w1a_t, w2_t, att_r)


def _edgemm(edge_attr, w1b_t):
    bm = 8000

    def body(ea_r, wt_r, v_r):
        v_r[...] = _dot(ea_r[...], wt_r[...])

    return pl.pallas_call(
        body,
        grid=(E_ // bm,),
        in_specs=[pl.BlockSpec((bm, 16), lambda i: (i, 0)), _full((16, H_))],
        out_specs=pl.BlockSpec((bm, H_), lambda i: (i, 0)),
        out_shape=jax.ShapeDtypeStruct((E_, H_), F32),
    )(edge_attr, w1b_t)


def _scoreprep(xcur, w_t, a_s, a_d):
    def body(x_r, wt_r, as_r, ad_r, xs_r, s1_r, s2_r):
        xs = _dot(x_r[...], wt_r[...])
        xs_r[...] = xs
        s1_r[...] = jnp.sum(xs * as_r[...], axis=1, keepdims=True)
        s2_r[...] = jnp.sum(xs * ad_r[...], axis=1, keepdims=True)

    bs_row = pl.BlockSpec((_BM, H_), lambda i: (i, 0))
    bs_one = pl.BlockSpec((_BM, 1), lambda i: (i, 0))
    return pl.pallas_call(
        body,
        grid=(N_ // _BM,),
        in_specs=[bs_row, _full((H_, H_)), _full((1, H_)), _full((1, H_))],
        out_specs=[bs_row, bs_one, bs_one],
        out_shape=[jax.ShapeDtypeStruct((N_, H_), F32),
                   jax.ShapeDtypeStruct((N_, 1), F32),
                   jax.ShapeDtypeStruct((N_, 1), F32)],
    )(xcur, w_t, a_s, a_d)


def _gru_math(hin, hprev, wih_t, whh_t, bih, bhh):
    gi = _dot(hin, wih_t) + bih
    gh = _dot(hprev, whh_t) + bhh
    r = jax.nn.sigmoid(gi[:, :H_] + gh[:, :H_])
    z = jax.nn.sigmoid(gi[:, H_:2 * H_] + gh[:, H_:2 * H_])
    n = jnp.tanh(gi[:, 2 * H_:] + r * gh[:, 2 * H_:])
    return jax.nn.relu((1.0 - z) * n + z * hprev)


def _gru_node(aggp, denp, bconv, hprev, wih_t, whh_t, bih, bhh):
    def body(a0_r, a1_r, d0_r, d1_r, b_r, hp_r, wi_r, wh_r, bi_r, bh_r, o_r):
        den = d0_r[0] + d1_r[0]
        hin = _elu((a0_r[0] + a1_r[0]) / (den + EPS) + b_r[...])
        o_r[...] = _gru_math(hin, hp_r[...], wi_r[...], wh_r[...],
                             bi_r[...], bh_r[...])

    bs_a0 = pl.BlockSpec((1, _BM, H_), lambda i: (0, i, 0))
    bs_a1 = pl.BlockSpec((1, _BM, H_), lambda i: (1, i, 0))
    bs_d0 = pl.BlockSpec((1, _BM, 1), lambda i: (0, i, 0))
    bs_d1 = pl.BlockSpec((1, _BM, 1), lambda i: (1, i, 0))
    bs_row = pl.BlockSpec((_BM, H_), lambda i: (i, 0))
    return pl.pallas_call(
        body,
        grid=(N_ // _BM,),
        in_specs=[bs_a0, bs_a1, bs_d0, bs_d1, _full((1, H_)), bs_row,
                  _full((H_, 3 * H_)), _full((H_, 3 * H_)),
                  _full((1, 3 * H_)), _full((1, 3 * H_))],
        out_specs=bs_row,
        out_shape=jax.ShapeDtypeStruct((N_, H_), F32),
    )(aggp, aggp, denp, denp, bconv, hprev, wih_t, whh_t, bih, bhh)


def _mol0(aggp, wad):
    def body(a0_r, a1_r, wad_r, o_r, s2_r):
        out = jax.nn.relu(a0_r[0] + a1_r[0])
        o_r[...] = out
        s2_r[...] = _dot(out, wad_r[...])

    bs_a0 = pl.BlockSpec((1, G_, H_), lambda i: (0, 0, 0))
    bs_a1 = pl.BlockSpec((1, G_, H_), lambda i: (1, 0, 0))
    return pl.pallas_call(
        body,
        grid=(1,),
        in_specs=[bs_a0, bs_a1, pl.BlockSpec((H_, 1), lambda i: (0, 0))],
        out_specs=[pl.BlockSpec((G_, H_), lambda i: (0, 0)),
                   pl.BlockSpec((G_, 1), lambda i: (0, 0))],
        out_shape=[jax.ShapeDtypeStruct((G_, H_), F32),
                   jax.ShapeDtypeStruct((G_, 1), F32)],
    )(aggp, aggp, wad)


def _gru_mol(aggp, denp, bconv, hprev, wih_t, whh_t, bih, bhh, wad):
    def body(a0_r, a1_r, d0_r, d1_r, b_r, hp_r, wi_r, wh_r, bi_r, bh_r,
             wad_r, o_r, s2_r):
        den = d0_r[0] + d1_r[0]
        hin = _elu((a0_r[0] + a1_r[0]) / (den + EPS) + b_r[...])
        out = _gru_math(hin, hp_r[...], wi_r[...], wh_r[...],
                        bi_r[...], bh_r[...])
        o_r[...] = out
        s2_r[...] = _dot(out, wad_r[...])

    bs_a0 = pl.BlockSpec((1, G_, H_), lambda i: (0, 0, 0))
    bs_a1 = pl.BlockSpec((1, G_, H_), lambda i: (1, 0, 0))
    bs_d0 = pl.BlockSpec((1, G_, 1), lambda i: (0, 0, 0))
    bs_d1 = pl.BlockSpec((1, G_, 1), lambda i: (1, 0, 0))
    bs_row = pl.BlockSpec((G_, H_), lambda i: (0, 0))
    return pl.pallas_call(
        body,
        grid=(1,),
        in_specs=[bs_a0, bs_a1, bs_d0, bs_d1,
                  pl.BlockSpec((1, H_), lambda i: (0, 0)), bs_row,
                  pl.BlockSpec((H_, 3 * H_), lambda i: (0, 0)),
                  pl.BlockSpec((H_, 3 * H_), lambda i: (0, 0)),
                  pl.BlockSpec((1, 3 * H_), lambda i: (0, 0)),
                  pl.BlockSpec((1, 3 * H_), lambda i: (0, 0)),
                  pl.BlockSpec((H_, 1), lambda i: (0, 0))],
        out_specs=[bs_row, pl.BlockSpec((G_, 1), lambda i: (0, 0))],
        out_shape=[jax.ShapeDtypeStruct((G_, H_), F32),
                   jax.ShapeDtypeStruct((G_, 1), F32)],
    )(aggp, aggp, denp, denp, bconv, hprev, wih_t, whh_t, bih, bhh, wad)


def _head(out, l2_t, l2_b, m1_t, m1_b, m2_t, m2_b):
    def body(o_r, w1_r, b1_r, w2_r, b2_r, w3_r, b3_r, res_r):
        t = _dot(o_r[...], w1_r[...]) + b1_r[...]
        t = _dot(jax.nn.relu(t), w2_r[...]) + b2_r[...]
        res_r[...] = _dot(jax.nn.relu(t), w3_r[...]) + b3_r[...]

    return pl.pallas_call(
        body,
        grid=(1,),
        in_specs=[pl.BlockSpec((G_, H_), lambda i: (0, 0)),
                  pl.BlockSpec((H_, 512), lambda i: (0, 0)),
                  pl.BlockSpec((1, 512), lambda i: (0, 0)),
                  pl.BlockSpec((512, 256), lambda i: (0, 0)),
                  pl.BlockSpec((1, 256), lambda i: (0, 0)),
                  pl.BlockSpec((256, 1), lambda i: (0, 0)),
                  pl.BlockSpec((1, 1), lambda i: (0, 0))],
        out_specs=pl.BlockSpec((G_, 1), lambda i: (0, 0)),
        out_shape=jax.ShapeDtypeStruct((G_, 1), F32),
    )(out, l2_t, l2_b, m1_t, m1_b, m2_t, m2_b)


@functools.lru_cache(maxsize=None)
def _mesh():
    return plsc.VectorSubcoreMesh(core_axis_name="c", subcore_axis_name="s",
                                  num_cores=NC, num_subcores=NS)


def _worker_id():
    return lax.axis_index("s") * NC + lax.axis_index("c")


@functools.lru_cache(maxsize=None)
def _sc_score(nsrc, ndst, ne):
    ew = ne // NW
    n1a = _ceil128(nsrc)
    n2a = _ceil128(ndst)

    @functools.partial(
        pl.kernel,
        out_type=(jax.ShapeDtypeStruct((ne,), F32),
                  jax.ShapeDtypeStruct((NW, L), F32)),
        mesh=_mesh(),
        compiler_params=pltpu.CompilerParams(needs_layout_passes=False),
        scratch_types=[pltpu.VMEM((n1a,), F32),
                       pltpu.VMEM((n2a,), F32),
                       pltpu.VMEM((ew,), jnp.int32),
                       pltpu.VMEM((ew,), jnp.int32),
                       pltpu.VMEM((ew,), F32),
                       pltpu.VMEM((L,), F32)],
    )
    def k(s1_h, s2_h, src_h, dst_h, a_h, wmax_h,
          s1_v, s2_v, src_v, dst_v, a_v, m_v):
        wid = _worker_id()
        base = wid * ew
        pltpu.sync_copy(s1_h, s1_v.at[pl.ds(0, nsrc)])
        pltpu.sync_copy(s2_h, s2_v.at[pl.ds(0, ndst)])
        pltpu.sync_copy(src_h.at[pl.ds(base, ew)], src_v)
        pltpu.sync_copy(dst_h.at[pl.ds(base, ew)], dst_v)

        def body(i, mx):
            o = i * L
            s_i = src_v[pl.ds(o, L)]
            d_i = dst_v[pl.ds(o, L)]
            v1 = plsc.load_gather(s1_v, [s_i])
            v2 = plsc.load_gather(s2_v, [d_i])
            a16 = _lrelu(v1 + v2)
            a_v[pl.ds(o, L)] = a16
            return jnp.maximum(mx, a16)

        mx = lax.fori_loop(0, ew // L, body,
                           jnp.full((L,), -1e30, F32))
        m_v[...] = mx
        pltpu.sync_copy(m_v, wmax_h.at[wid])
        pltpu.sync_copy(a_v, a_h.at[pl.ds(base, ew)])

    return k


@functools.lru_cache(maxsize=None)
def _sc_gatescore(nsrc, ne):
    ew = ne // NW
    nch = ew // CH
    n1a = _ceil128(nsrc)

    @functools.partial(
        pl.kernel,
        out_type=(jax.ShapeDtypeStruct((ne,), F32),
                  jax.ShapeDtypeStruct((NW, L), F32)),
        mesh=_mesh(),
        compiler_params=pltpu.CompilerParams(needs_layout_passes=False),
        scratch_types=[pltpu.VMEM((n1a,), F32),
                       pltpu.VMEM((H_,), F32),
                       pltpu.VMEM((CH, H_), F32),
                       pltpu.VMEM((CH, H_), F32),
                       pltpu.VMEM((CH,), jnp.int32),
                       pltpu.VMEM((ew,), jnp.int32),
                       pltpu.VMEM((ew,), F32),
                       pltpu.VMEM((L,), F32),
                       pltpu.VMEM((128,), F32),
                       pltpu.SemaphoreType.DMA],
    )
    def k(u_h, v_h, r_h, attl_h, src_h, dst_h, a_h, wmax_h,
          r_v, attl_v, urows, vrows, sbuf, dst_v, a_v, m_v, red_v, sem):
        wid = _worker_id()
        base = wid * ew
        pltpu.sync_copy(r_h, r_v.at[pl.ds(0, nsrc)])
        pltpu.sync_copy(attl_h, attl_v)
        pltpu.sync_copy(dst_h.at[pl.ds(base, ew)], dst_v)

        def chunk(ch, mx):
            gb = base + ch * CH
            pltpu.sync_copy(src_h.at[pl.ds(gb, CH)], sbuf)
            pltpu.async_copy(u_h.at[sbuf], urows, sem).wait()
            pltpu.sync_copy(v_h.at[pl.ds(gb, CH)], vrows)
            lo = ch * CH
            lanes = jax.lax.iota(jnp.int32, L)
            for g in range(CH // L):
                def jbody(j2, t16):
                    row = g * L + j2
                    acc = jnp.zeros((L,), F32)
                    for q in range(H_ // L):
                        m = _lrelu(urows[row, pl.ds(q * L, L)]
                                   + vrows[row, pl.ds(q * L, L)])
                        acc = acc + m * attl_v[pl.ds(q * L, L)]
                    tot = _lane_reduce(acc, red_v, jnp.add)
                    return t16 + jnp.where(lanes == j2, tot, 0.0)

                t16 = lax.fori_loop(0, L, jbody, jnp.zeros((L,), F32))
                d_i = dst_v[pl.ds(lo + g * L, L)]
                rd = plsc.load_gather(r_v, [d_i])
                a16 = _lrelu(t16 + rd)
                a_v[pl.ds(lo + g * L, L)] = a16
                mx = jnp.maximum(mx, a16)
            return mx

        mx = lax.fori_loop(0, nch, chunk,
                           jnp.full((L,), -1e30, F32))
        m_v[...] = mx
        pltpu.sync_copy(m_v, wmax_h.at[wid])
        pltpu.sync_copy(a_v, a_h.at[pl.ds(base, ew)])

    return k


@functools.lru_cache(maxsize=None)
def _sc_aggregate(nsrc, nda, ne):
    ew = ne // NW
    nch = ew // CH
    rpt = nda // NS

    @functools.partial(
        pl.kernel,
        out_type=(jax.ShapeDtypeStruct((2 * nda, H_), F32),
                  jax.ShapeDtypeStruct((2 * nda,), F32)),
        mesh=_mesh(),
        compiler_params=pltpu.CompilerParams(needs_layout_passes=False),
        scratch_types=[pltpu.VMEM((ew,), F32),
                       pltpu.VMEM((128,), F32),
                       pltpu.VMEM((CH, H_), F32),
                       pltpu.VMEM((CH,), jnp.int32),
                       pltpu.VMEM((CH,), jnp.int32),
                       pltpu.VMEM((NW, L), F32),
                       pltpu.VMEM((128,), F32),
                       pltpu.VMEM_SHARED((nda, H_), F32),
                       pltpu.VMEM_SHARED((nda,), F32),
                       pltpu.SemaphoreType.DMA],
    )
    def k(a_h, wmax_h, xs_h, src_h, dst_h, outp_h, denp_h,
          a_v, pbuf, rows, sbuf, dbuf, wm_v, red_v, out_sh, den_sh, sem):
        cid = lax.axis_index("c")
        sid = lax.axis_index("s")
        wid = sid * NC + cid
        base = wid * ew
        off = sid * rpt

        def zrow(j, _):
            for q in range(H_ // L):
                rows[j, pl.ds(q * L, L)] = jnp.zeros((L,), F32)
            return 0

        lax.fori_loop(0, CH, zrow, 0)
        for q in range(CH // L):
            pbuf[pl.ds(q * L, L)] = jnp.zeros((L,), F32)
        nz = rpt // CH
        for t in range(nz):
            pltpu.sync_copy(rows, out_sh.at[pl.ds(off + t * CH, CH)])
            pltpu.sync_copy(pbuf.at[pl.ds(0, CH)],
                            den_sh.at[pl.ds(off + t * CH, CH)])
        rem = rpt % CH
        if rem:
            pltpu.sync_copy(rows.at[pl.ds(0, rem)],
                            out_sh.at[pl.ds(off + nz * CH, rem)])
            pltpu.sync_copy(pbuf.at[pl.ds(0, rem)],
                            den_sh.at[pl.ds(off + nz * CH, rem)])
        plsc.subcore_barrier()

        pltpu.sync_copy(wmax_h, wm_v)

        def wm(i, mx):
            return jnp.maximum(mx, wm_v[i, :])

        gmax = _lane_reduce(lax.fori_loop(0, NW, wm,
                                          jnp.full((L,), -1e30, F32)),
                            red_v, jnp.maximum)
        pltpu.sync_copy(a_h.at[pl.ds(base, ew)], a_v)

        def chunk(ch, _):
            gb = base + ch * CH
            pltpu.sync_copy(src_h.at[pl.ds(gb, CH)], sbuf)
            pltpu.sync_copy(dst_h.at[pl.ds(gb, CH)], dbuf)
            pltpu.async_copy(xs_h.at[sbuf], rows, sem).wait()
            lo = ch * CH
            for q in range(CH // L):
                av = a_v[pl.ds(lo + q * L, L)]
                pbuf[pl.ds(q * L, L)] = jnp.exp(av - gmax)

            def scale(j, _):
                pv = plsc.load_gather(pbuf, [jnp.full((L,), j, jnp.int32)])
                for q in range(H_ // L):
                    rows[j, pl.ds(q * L, L)] = rows[j, pl.ds(q * L, L)] * pv
                return 0

            lax.fori_loop(0, CH, scale, 0)
            pltpu.sync_copy(rows, out_sh.at[dbuf], add=True)
            pltpu.sync_copy(pbuf.at[pl.ds(0, CH)], den_sh.at[dbuf], add=True)
            return 0

        lax.fori_loop(0, nch, chunk, 0)
        plsc.subcore_barrier()

        ob = cid * nda + off
        for t in range(nz):
            pltpu.sync_copy(out_sh.at[pl.ds(off + t * CH, CH)], rows)
            pltpu.sync_copy(rows, outp_h.at[pl.ds(ob + t * CH, CH)])
            pltpu.sync_copy(den_sh.at[pl.ds(off + t * CH, CH)],
                            pbuf.at[pl.ds(0, CH)])
            pltpu.sync_copy(pbuf.at[pl.ds(0, CH)],
                            denp_h.at[pl.ds(ob + t * CH, CH)])
        if rem:
            pltpu.sync_copy(out_sh.at[pl.ds(off + nz * CH, rem)],
                            rows.at[pl.ds(0, rem)])
            pltpu.sync_copy(rows.at[pl.ds(0, rem)],
                            outp_h.at[pl.ds(ob + nz * CH, rem)])
            pltpu.sync_copy(den_sh.at[pl.ds(off + nz * CH, rem)],
                            pbuf.at[pl.ds(0, rem)])
            pltpu.sync_copy(pbuf.at[pl.ds(0, rem)],
                            denp_h.at[pl.ds(ob + nz * CH, rem)])

    return k


def kernel(x, edge_attr, params, edge_index, batch):
    p = params
    src = edge_index[0]
    dst = edge_index[1]

    lin1_wt = p["lin1_W"].T
    lin1_b = p["lin1_b"][None, :]
    w1a_t = p["gc_W1"][:, :H_].T
    w1b_t = p["gc_W1"][:, H_:].T
    w2_t = p["gc_W2"].T
    att_r = p["gc_att_r"][None, :]

    h0, u, w2, r_n = _prep0(x, lin1_wt, lin1_b, w1a_t, w2_t, att_r)
    v = _edgemm(edge_attr, w1b_t)

    a_gc, wmax = _sc_gatescore(N_, E_)(
        u, v, r_n.reshape(-1), p["gc_att_l"], src, dst)
    outp, denp = _sc_aggregate(N_, NA_N, E_)(a_gc, wmax, w2, src, dst)
    aggp = outp.reshape(2, NA_N, H_)
    denp3 = denp.reshape(2, NA_N, 1)

    g = p["gru0"]
    xcur = _gru_node(aggp, denp3, p["gc_b"][None, :], h0,
                     g["Wih"].T, g["Whh"].T, g["bih"][None, :],
                     g["bhh"][None, :])

    for name_c, name_g in (("conv1", "gru1"), ("conv2", "gru2")):
        c = p[name_c]
        xs, s1, s2 = _scoreprep(xcur, c["W"].T, c["as"][None, :],
                                c["ad"][None, :])
        a_e, wmax = _sc_score(N_, N_, E_)(
            s1.reshape(-1), s2.reshape(-1), src, dst)
        outp, denp = _sc_aggregate(N_, NA_N, E_)(a_e, wmax, xs, src, dst)
        g = p[name_g]
        xcur = _gru_node(outp.reshape(2, NA_N, H_),
                         denp.reshape(2, NA_N, 1), c["b"][None, :], xcur,
                         g["Wih"].T, g["Whh"].T, g["bih"][None, :],
                         g["bhh"][None, :])

    c = p["mol"]
    xs, s1, _ = _scoreprep(xcur, c["W"].T, c["as"][None, :],
                           c["ad"][None, :])
    wad = (c["W"].T @ c["ad"])[:, None]
    npad = NE_M - N_
    srcm = jnp.concatenate([jnp.arange(N_, dtype=jnp.int32),
                            jnp.zeros((npad,), jnp.int32)])
    dstm = jnp.concatenate([batch, jnp.full((npad,), G_, jnp.int32)])

    a0 = jnp.zeros((NE_M,), F32)
    wm0 = jnp.zeros((NW, L), F32)
    outp, _ = _sc_aggregate(N_, NA_G, NE_M)(a0, wm0, xcur, srcm, dstm)
    out, s2g = _mol0(outp.reshape(2, NA_G, H_), wad)

    g = p["molgru"]
    s1f = s1.reshape(-1)
    for _ in range(3):
        s2pad = jnp.concatenate([s2g.reshape(-1),
                                 jnp.zeros((NA_G - G_,), F32)])
        a_e, wmax = _sc_score(N_, NA_G, NE_M)(s1f, s2pad, srcm, dstm)
        outp, denp = _sc_aggregate(N_, NA_G, NE_M)(a_e, wmax, xs, srcm, dstm)
        out, s2g = _gru_mol(outp.reshape(2, NA_G, H_),
                            denp.reshape(2, NA_G, 1), c["b"][None, :], out,
                            g["Wih"].T, g["Whh"].T, g["bih"][None, :],
                            g["bhh"][None, :], wad)

    res = _head(out, p["lin2_W"].T, p["lin2_b"][None, :],
                p["mlp1_W"].T, p["mlp1_b"][None, :],
                p["mlp2_W"].T, p["mlp2_b"][None, :])
    return res.reshape(-1)
